# baseline (speedup 1.0000x reference)
import functools
import math

import jax
import jax.numpy as jnp
import numpy as np
from jax import lax
from jax.experimental import pallas as pl
from jax.experimental.pallas import tpu as pltpu
from jax.experimental.pallas import tpu_sc as plsc

VOCAB_SIZE = 32000
MODEL_DIM = 512
MAX_SEQ_LENGTH = 2048
SCALE = math.sqrt(MODEL_DIM)

NUM_CORES = 2
NUM_SUBCORES = 16
NUM_WORKERS = NUM_CORES * NUM_SUBCORES

BATCH = 4
SEQ = 2048
TOTAL_ROWS = BATCH * SEQ
NUM_HALVES = 2
HALF_ROWS = TOTAL_ROWS // NUM_HALVES
ROWS_PER_WORKER = HALF_ROWS // NUM_WORKERS
CHUNK_ROWS = 64
NUM_CHUNKS = ROWS_PER_WORKER // CHUNK_ROWS

ROW_BLOCK = 512
HALF_ROW_BLOCKS = HALF_ROWS // ROW_BLOCK
PE_BLOCKS = SEQ // ROW_BLOCK


def _pos_encoding_np(max_seq_length, model_dim):
    position = np.arange(max_seq_length)[:, None].astype(np.float32)
    div_term = np.exp(
        np.arange(0, model_dim, 2).astype(np.float32)
        * (-math.log(10000.0) / model_dim)
    )
    pe = np.zeros((max_seq_length, model_dim), dtype=np.float32)
    pe[:, 0::2] = np.sin(position * div_term)
    pe[:, 1::2] = np.cos(position * div_term)
    return pe


_PE = _pos_encoding_np(MAX_SEQ_LENGTH, MODEL_DIM)


def _sc_gather_body(half_base, idx_hbm, table_hbm, out_hbm, idx_v, rows0, rows1,
                    sem_g0, sem_g1, sem_s0, sem_s1):
    wid = lax.axis_index("s") * NUM_CORES + lax.axis_index("c")
    base = wid * ROWS_PER_WORKER

    rows = [rows0, rows1]
    sem_g = [sem_g0, sem_g1]
    sem_s = [sem_s0, sem_s1]

    pltpu.sync_copy(idx_hbm.at[pl.ds(half_base + base, ROWS_PER_WORKER)], idx_v)

    def fire_gather(c):
        b = c % 2
        return pltpu.async_copy(
            table_hbm.at[idx_v.at[pl.ds(c * CHUNK_ROWS, CHUNK_ROWS)]],
            rows[b], sem_g[b])

    pending = {0: fire_gather(0)}
    stores = {}

    for c in range(NUM_CHUNKS):
        b = c % 2
        if c - 1 in stores:
            stores.pop(c - 1).wait()
        if c + 1 < NUM_CHUNKS:
            pending[c + 1] = fire_gather(c + 1)
        pending.pop(c).wait()
        stores[c] = pltpu.async_copy(
            rows[b], out_hbm.at[pl.ds(base + c * CHUNK_ROWS, CHUNK_ROWS)],
            sem_s[b])

    for c in sorted(stores):
        stores[c].wait()


def _fma_kernel(g_ref, pe_ref, o_ref):
    o_ref[...] = g_ref[...] * SCALE + pe_ref[...]


def _fma_half_kernel(g_ref, pe_ref, _, o_ref):
    o_ref[...] = g_ref[...] * SCALE + pe_ref[...]


def _sc_gather(x_flat, table, half_base):
    mesh = plsc.VectorSubcoreMesh(
        core_axis_name="c", subcore_axis_name="s",
        num_cores=NUM_CORES, num_subcores=NUM_SUBCORES)
    return pl.kernel(
        functools.partial(_sc_gather_body, half_base),
        out_type=jax.ShapeDtypeStruct((HALF_ROWS, MODEL_DIM), jnp.float32),
        mesh=mesh,
        scratch_types=[
            pltpu.VMEM((ROWS_PER_WORKER,), jnp.int32),
            pltpu.VMEM((CHUNK_ROWS, MODEL_DIM), jnp.float32),
            pltpu.VMEM((CHUNK_ROWS, MODEL_DIM), jnp.float32),
            pltpu.SemaphoreType.DMA,
            pltpu.SemaphoreType.DMA,
            pltpu.SemaphoreType.DMA,
            pltpu.SemaphoreType.DMA,
        ],
    )(x_flat, table)


@jax.jit
def _embed(x, table):
    x_flat = x.reshape(TOTAL_ROWS).astype(jnp.int32)
    pe = jnp.asarray(_PE)

    g0 = _sc_gather(x_flat, table, 0)
    g1 = _sc_gather(x_flat, table, HALF_ROWS)

    partial = pl.pallas_call(
        _fma_kernel,
        out_shape=jax.ShapeDtypeStruct((TOTAL_ROWS, MODEL_DIM), jnp.float32),
        grid=(HALF_ROW_BLOCKS,),
        in_specs=[
            pl.BlockSpec((ROW_BLOCK, MODEL_DIM), lambda i: (i, 0)),
            pl.BlockSpec((ROW_BLOCK, MODEL_DIM), lambda i: (i % PE_BLOCKS, 0)),
        ],
        out_specs=pl.BlockSpec((ROW_BLOCK, MODEL_DIM), lambda i: (i, 0)),
        compiler_params=pltpu.CompilerParams(
            dimension_semantics=("arbitrary",),
        ),
    )(g0, pe)

    out = pl.pallas_call(
        _fma_half_kernel,
        out_shape=jax.ShapeDtypeStruct((TOTAL_ROWS, MODEL_DIM), jnp.float32),
        grid=(HALF_ROW_BLOCKS,),
        in_specs=[
            pl.BlockSpec((ROW_BLOCK, MODEL_DIM), lambda i: (i, 0)),
            pl.BlockSpec((ROW_BLOCK, MODEL_DIM), lambda i: (i % PE_BLOCKS, 0)),
            pl.BlockSpec(memory_space=pl.ANY),
        ],
        out_specs=pl.BlockSpec(
            (ROW_BLOCK, MODEL_DIM), lambda i: (i + HALF_ROW_BLOCKS, 0)),
        input_output_aliases={2: 0},
        compiler_params=pltpu.CompilerParams(
            dimension_semantics=("arbitrary",),
        ),
    )(g1, pe, partial)
    return out.reshape(BATCH, SEQ, MODEL_DIM)


def kernel(x, table):
    return _embed(x, table)


# speedup vs baseline: 1.1104x; 1.1104x over previous
import functools
import math

import jax
import jax.numpy as jnp
import numpy as np
from jax import lax
from jax.experimental import pallas as pl
from jax.experimental.pallas import tpu as pltpu
from jax.experimental.pallas import tpu_sc as plsc

VOCAB_SIZE = 32000
MODEL_DIM = 512
MAX_SEQ_LENGTH = 2048
SCALE = math.sqrt(MODEL_DIM)

NUM_CORES = 2
NUM_SUBCORES = 16
NUM_WORKERS = NUM_CORES * NUM_SUBCORES

BATCH = 4
SEQ = 2048
TOTAL_ROWS = BATCH * SEQ
NUM_HALVES = 2
HALF_ROWS = TOTAL_ROWS // NUM_HALVES
ROWS_PER_WORKER = HALF_ROWS // NUM_WORKERS
CHUNK_ROWS = 64
NUM_CHUNKS = ROWS_PER_WORKER // CHUNK_ROWS

ROW_BLOCK = 2048
HALF_ROW_BLOCKS = HALF_ROWS // ROW_BLOCK
PE_BLOCKS = SEQ // ROW_BLOCK


def _pos_encoding_np(max_seq_length, model_dim):
    position = np.arange(max_seq_length)[:, None].astype(np.float32)
    div_term = np.exp(
        np.arange(0, model_dim, 2).astype(np.float32)
        * (-math.log(10000.0) / model_dim)
    )
    pe = np.zeros((max_seq_length, model_dim), dtype=np.float32)
    pe[:, 0::2] = np.sin(position * div_term)
    pe[:, 1::2] = np.cos(position * div_term)
    return pe


_PE = _pos_encoding_np(MAX_SEQ_LENGTH, MODEL_DIM)


def _sc_gather_body(half_base, idx_hbm, table_hbm, out_hbm, idx_v, rows0, rows1,
                    sem_g0, sem_g1, sem_s0, sem_s1):
    wid = lax.axis_index("s") * NUM_CORES + lax.axis_index("c")
    base = wid * ROWS_PER_WORKER

    rows = [rows0, rows1]
    sem_g = [sem_g0, sem_g1]
    sem_s = [sem_s0, sem_s1]

    pltpu.sync_copy(idx_hbm.at[pl.ds(half_base + base, ROWS_PER_WORKER)], idx_v)

    def fire_gather(c):
        b = c % 2
        return pltpu.async_copy(
            table_hbm.at[idx_v.at[pl.ds(c * CHUNK_ROWS, CHUNK_ROWS)]],
            rows[b], sem_g[b])

    pending = {0: fire_gather(0)}
    stores = {}

    for c in range(NUM_CHUNKS):
        b = c % 2
        if c - 1 in stores:
            stores.pop(c - 1).wait()
        if c + 1 < NUM_CHUNKS:
            pending[c + 1] = fire_gather(c + 1)
        pending.pop(c).wait()
        stores[c] = pltpu.async_copy(
            rows[b], out_hbm.at[pl.ds(base + c * CHUNK_ROWS, CHUNK_ROWS)],
            sem_s[b])

    for c in sorted(stores):
        stores[c].wait()


def _fma_kernel(g_ref, pe_ref, o_ref):
    o_ref[...] = g_ref[...] * SCALE + pe_ref[...]


def _fma_half_kernel(g_ref, pe_ref, _, o_ref):
    o_ref[...] = g_ref[...] * SCALE + pe_ref[...]


def _sc_gather(x_flat, table, half_base):
    mesh = plsc.VectorSubcoreMesh(
        core_axis_name="c", subcore_axis_name="s",
        num_cores=NUM_CORES, num_subcores=NUM_SUBCORES)
    return pl.kernel(
        functools.partial(_sc_gather_body, half_base),
        out_type=jax.ShapeDtypeStruct((HALF_ROWS, MODEL_DIM), jnp.float32),
        mesh=mesh,
        scratch_types=[
            pltpu.VMEM((ROWS_PER_WORKER,), jnp.int32),
            pltpu.VMEM((CHUNK_ROWS, MODEL_DIM), jnp.float32),
            pltpu.VMEM((CHUNK_ROWS, MODEL_DIM), jnp.float32),
            pltpu.SemaphoreType.DMA,
            pltpu.SemaphoreType.DMA,
            pltpu.SemaphoreType.DMA,
            pltpu.SemaphoreType.DMA,
        ],
    )(x_flat, table)


@jax.jit
def _embed(x, table):
    x_flat = x.reshape(TOTAL_ROWS).astype(jnp.int32)
    pe = jnp.asarray(_PE)

    g0 = _sc_gather(x_flat, table, 0)
    g1 = _sc_gather(x_flat, table, HALF_ROWS)

    partial = pl.pallas_call(
        _fma_kernel,
        out_shape=jax.ShapeDtypeStruct((TOTAL_ROWS, MODEL_DIM), jnp.float32),
        grid=(HALF_ROW_BLOCKS,),
        in_specs=[
            pl.BlockSpec((ROW_BLOCK, MODEL_DIM), lambda i: (i, 0)),
            pl.BlockSpec((ROW_BLOCK, MODEL_DIM), lambda i: (i % PE_BLOCKS, 0)),
        ],
        out_specs=pl.BlockSpec((ROW_BLOCK, MODEL_DIM), lambda i: (i, 0)),
        compiler_params=pltpu.CompilerParams(
            dimension_semantics=("arbitrary",),
        ),
    )(g0, pe)

    out = pl.pallas_call(
        _fma_half_kernel,
        out_shape=jax.ShapeDtypeStruct((TOTAL_ROWS, MODEL_DIM), jnp.float32),
        grid=(HALF_ROW_BLOCKS,),
        in_specs=[
            pl.BlockSpec((ROW_BLOCK, MODEL_DIM), lambda i: (i, 0)),
            pl.BlockSpec((ROW_BLOCK, MODEL_DIM), lambda i: (i % PE_BLOCKS, 0)),
            pl.BlockSpec(memory_space=pl.ANY),
        ],
        out_specs=pl.BlockSpec(
            (ROW_BLOCK, MODEL_DIM), lambda i: (i + HALF_ROW_BLOCKS, 0)),
        input_output_aliases={2: 0},
        compiler_params=pltpu.CompilerParams(
            dimension_semantics=("arbitrary",),
        ),
    )(g1, pe, partial)
    return out.reshape(BATCH, SEQ, MODEL_DIM)


def kernel(x, table):
    return _embed(x, table)
